# Initial kernel scaffold; baseline (speedup 1.0000x reference)
import jax
import jax.numpy as jnp
from jax.experimental import pallas as pl


def kernel(x_p2, x_p3, x_p4, x_p5, boxes0, boxes1):
    raise NotImplementedError("write your pallas kernel here")


# speedup vs baseline: 6.3174x; 6.3174x over previous
import functools

import jax
import jax.numpy as jnp
from jax import lax
from jax.experimental import pallas as pl
from jax.experimental.pallas import tpu as pltpu
from jax.experimental.pallas import tpu_sc as plsc

_OS = 7
_SR = 2
_N = 512
_C = 256
_NS = _OS * _OS * _SR * _SR
_RPR = _NS * 4
_NCHUNK = 7
_CHUNK = _RPR // _NCHUNK
_NW = 32
_RPW = _N // _NW
_EPS = 2.220446049250313e-16

_OFFS = (0, 131072, 163840, 172032)
_HS = (256, 128, 64, 32)


def _grid_body(b0_ref, b1_ref, i00, i01, i10, i11, w00, w01, w10, w11):
    boxes = jnp.concatenate([b0_ref[...], b1_ref[...]], axis=0)
    c0 = boxes[:, 0:1]
    c1 = boxes[:, 1:2]
    c2 = boxes[:, 2:3]
    c3 = boxes[:, 3:4]
    t = jnp.sqrt((c2 - c0) * (c3 - c1)) / 224.0 + _EPS
    lvl = ((t >= 0.5).astype(jnp.int32) + (t >= 1.0).astype(jnp.int32)
           + (t >= 2.0).astype(jnp.int32))

    def bylvl(v0, v1, v2, v3, dtype):
        return jnp.where(
            lvl == 0, jnp.array(v0, dtype),
            jnp.where(lvl == 1, jnp.array(v1, dtype),
                      jnp.where(lvl == 2, jnp.array(v2, dtype),
                                jnp.array(v3, dtype))))

    scale = bylvl(0.25, 0.125, 0.0625, 0.03125, jnp.float32)
    hi = bylvl(*_HS, jnp.int32)
    off = bylvl(*_OFFS, jnp.int32)
    hw = bylvl(256 * 256, 128 * 128, 64 * 64, 32 * 32, jnp.int32)
    ridx = lax.broadcasted_iota(jnp.int32, (_N, 1), 0)
    base = off + jnp.where(ridx >= 256, hw, 0)

    x1 = c0 * scale
    y1 = c1 * scale
    x2 = c2 * scale
    y2 = c3 * scale
    bin_w = jnp.maximum(x2 - x1, 1.0) / _OS
    bin_h = jnp.maximum(y2 - y1, 1.0) / _OS

    s = lax.broadcasted_iota(jnp.int32, (_N, _NS), 1)
    phf = (s // 28).astype(jnp.float32)
    pwf = ((s // 4) % _OS).astype(jnp.float32)
    iyf = ((s % 4) // 2).astype(jnp.float32)
    ixf = (s % 2).astype(jnp.float32)

    y = y1 + phf * bin_h + (iyf + 0.5) * bin_h / _SR
    x = x1 + pwf * bin_w + (ixf + 0.5) * bin_w / _SR
    hf = hi.astype(jnp.float32)
    valid = (y > -1.0) & (y < hf) & (x > -1.0) & (x < hf)
    y = jnp.clip(y, 0.0, hf - 1.0)
    x = jnp.clip(x, 0.0, hf - 1.0)
    y0f = jnp.floor(y)
    x0f = jnp.floor(x)
    ly = y - y0f
    lx = x - x0f
    y0 = y0f.astype(jnp.int32)
    x0 = x0f.astype(jnp.int32)
    yb = jnp.minimum(y0 + 1, hi - 1)
    xb = jnp.minimum(x0 + 1, hi - 1)
    vw = jnp.where(valid, 0.25, 0.0)
    w00[...] = (1.0 - ly) * (1.0 - lx) * vw
    w01[...] = (1.0 - ly) * lx * vw
    w10[...] = ly * (1.0 - lx) * vw
    w11[...] = ly * lx * vw
    i00[...] = base + y0 * hi + x0
    i01[...] = base + y0 * hi + xb
    i10[...] = base + yb * hi + x0
    i11[...] = base + yb * hi + xb


def _make_grid():
    sd = jax.ShapeDtypeStruct
    return pl.pallas_call(
        _grid_body,
        out_shape=(
            sd((_N, _NS), jnp.int32), sd((_N, _NS), jnp.int32),
            sd((_N, _NS), jnp.int32), sd((_N, _NS), jnp.int32),
            sd((_N, _NS), jnp.float32), sd((_N, _NS), jnp.float32),
            sd((_N, _NS), jnp.float32), sd((_N, _NS), jnp.float32),
        ),
    )


def _pool_body(table_hbm, idx_hbm, w_hbm, out_hbm,
               idx_v, w_v, rows0, rows1, acc, sem0, sem1):
    wid = lax.axis_index("s") * 2 + lax.axis_index("c")

    def roi_body(i, carry):
        n = wid * _RPW + i
        pltpu.sync_copy(idx_hbm.at[n], idx_v)
        pltpu.sync_copy(w_hbm.at[n], w_v)
        handle0 = pltpu.async_copy(table_hbm.at[idx_v.at[0]], rows0, sem0)

        def zero_body(b, c2_):
            for cc in range(16):
                acc[b, pl.ds(cc * 16, 16)] = jnp.zeros((16,), jnp.float32)
            return c2_
        lax.fori_loop(0, _OS * _OS, zero_body, 0)

        handles = [handle0, None]
        for k in range(_NCHUNK):
            cur = k % 2
            handles[cur].wait()
            if k + 1 < _NCHUNK:
                nxt = (k + 1) % 2
                handles[nxt] = pltpu.async_copy(
                    table_hbm.at[idx_v.at[k + 1]],
                    rows1 if nxt else rows0,
                    sem1 if nxt else sem0)
            buf = rows1 if cur else rows0

            def proc_body(j, c3_, _k=k, _buf=buf):
                r = _k * _CHUNK + j
                b = r // 16
                wv = w_v[pl.ds(r * 16, 16)]
                for cc in range(16):
                    sl = pl.ds(cc * 16, 16)
                    acc[b, sl] = acc[b, sl] + wv * _buf[j, sl]
                return c3_
            lax.fori_loop(0, _CHUNK, proc_body, 0)

        pltpu.sync_copy(acc, out_hbm.at[n])
        return carry
    lax.fori_loop(0, _RPW, roi_body, 0)


def _make_pool():
    return functools.partial(
        pl.kernel,
        mesh=plsc.VectorSubcoreMesh(core_axis_name="c", subcore_axis_name="s"),
        out_type=jax.ShapeDtypeStruct((_N, _OS * _OS, _C), jnp.float32),
        scratch_types=[
            pltpu.VMEM((_NCHUNK, _CHUNK), jnp.int32),
            pltpu.VMEM((_RPR * 16,), jnp.float32),
            pltpu.VMEM((_CHUNK, _C), jnp.float32),
            pltpu.VMEM((_CHUNK, _C), jnp.float32),
            pltpu.VMEM((_OS * _OS, _C), jnp.float32),
            pltpu.SemaphoreType.DMA,
            pltpu.SemaphoreType.DMA,
        ],
    )(_pool_body)


def _rows(x):
    b, c, h, w = x.shape
    return jnp.transpose(x, (0, 2, 3, 1)).reshape(b * h * w, c)


def kernel(x_p2, x_p3, x_p4, x_p5, boxes0, boxes1):
    table = jnp.concatenate(
        [_rows(x_p2), _rows(x_p3), _rows(x_p4), _rows(x_p5)], axis=0)
    i00, i01, i10, i11, w00, w01, w10, w11 = _make_grid()(boxes0, boxes1)
    idx = jnp.stack([i00, i01, i10, i11], axis=-1).reshape(_N, _NCHUNK, _CHUNK)
    w = jnp.stack([w00, w01, w10, w11], axis=-1).reshape(_N, _RPR)
    w = jnp.broadcast_to(w[:, :, None], (_N, _RPR, 16)).reshape(_N, _RPR * 16)
    out = _make_pool()(table, idx, w)
    return jnp.transpose(out.reshape(_N, _OS, _OS, _C), (0, 3, 1, 2))


# speedup vs baseline: 15.6013x; 2.4696x over previous
import functools

import jax
import jax.numpy as jnp
from jax import lax
from jax.experimental import pallas as pl
from jax.experimental.pallas import tpu as pltpu
from jax.experimental.pallas import tpu_sc as plsc

_OS = 7
_SR = 2
_N = 512
_C = 256
_NS = _OS * _OS * _SR * _SR
_RPR = _NS * 4
_NCHUNK = 7
_CHUNK = _RPR // _NCHUNK
_NW = 32
_RPW = _N // _NW
_EPS = 2.220446049250313e-16

_OFFS = (0, 131072, 163840, 172032)
_HS = (256, 128, 64, 32)


def _grid_body(b0_ref, b1_ref, i00, i01, i10, i11, w00, w01, w10, w11):
    boxes = jnp.concatenate([b0_ref[...], b1_ref[...]], axis=0)
    c0 = boxes[:, 0:1]
    c1 = boxes[:, 1:2]
    c2 = boxes[:, 2:3]
    c3 = boxes[:, 3:4]
    t = jnp.sqrt((c2 - c0) * (c3 - c1)) / 224.0 + _EPS
    lvl = ((t >= 0.5).astype(jnp.int32) + (t >= 1.0).astype(jnp.int32)
           + (t >= 2.0).astype(jnp.int32))

    def bylvl(v0, v1, v2, v3, dtype):
        return jnp.where(
            lvl == 0, jnp.array(v0, dtype),
            jnp.where(lvl == 1, jnp.array(v1, dtype),
                      jnp.where(lvl == 2, jnp.array(v2, dtype),
                                jnp.array(v3, dtype))))

    scale = bylvl(0.25, 0.125, 0.0625, 0.03125, jnp.float32)
    hi = bylvl(*_HS, jnp.int32)
    off = bylvl(*_OFFS, jnp.int32)
    hw = bylvl(256 * 256, 128 * 128, 64 * 64, 32 * 32, jnp.int32)
    ridx = lax.broadcasted_iota(jnp.int32, (_N, 1), 0)
    base = off + jnp.where(ridx >= 256, hw, 0)

    x1 = c0 * scale
    y1 = c1 * scale
    x2 = c2 * scale
    y2 = c3 * scale
    bin_w = jnp.maximum(x2 - x1, 1.0) / _OS
    bin_h = jnp.maximum(y2 - y1, 1.0) / _OS

    s = lax.broadcasted_iota(jnp.int32, (_N, _NS), 1)
    phf = (s // 28).astype(jnp.float32)
    pwf = ((s // 4) % _OS).astype(jnp.float32)
    iyf = ((s % 4) // 2).astype(jnp.float32)
    ixf = (s % 2).astype(jnp.float32)

    y = y1 + phf * bin_h + (iyf + 0.5) * bin_h / _SR
    x = x1 + pwf * bin_w + (ixf + 0.5) * bin_w / _SR
    hf = hi.astype(jnp.float32)
    valid = (y > -1.0) & (y < hf) & (x > -1.0) & (x < hf)
    y = jnp.clip(y, 0.0, hf - 1.0)
    x = jnp.clip(x, 0.0, hf - 1.0)
    y0f = jnp.floor(y)
    x0f = jnp.floor(x)
    ly = y - y0f
    lx = x - x0f
    y0 = y0f.astype(jnp.int32)
    x0 = x0f.astype(jnp.int32)
    yb = jnp.minimum(y0 + 1, hi - 1)
    xb = jnp.minimum(x0 + 1, hi - 1)
    vw = jnp.where(valid, 0.25, 0.0)
    w00[...] = (1.0 - ly) * (1.0 - lx) * vw
    w01[...] = (1.0 - ly) * lx * vw
    w10[...] = ly * (1.0 - lx) * vw
    w11[...] = ly * lx * vw
    i00[...] = base + y0 * hi + x0
    i01[...] = base + y0 * hi + xb
    i10[...] = base + yb * hi + x0
    i11[...] = base + yb * hi + xb


def _make_grid():
    sd = jax.ShapeDtypeStruct
    return pl.pallas_call(
        _grid_body,
        out_shape=(
            sd((_N, _NS), jnp.int32), sd((_N, _NS), jnp.int32),
            sd((_N, _NS), jnp.int32), sd((_N, _NS), jnp.int32),
            sd((_N, _NS), jnp.float32), sd((_N, _NS), jnp.float32),
            sd((_N, _NS), jnp.float32), sd((_N, _NS), jnp.float32),
        ),
    )


def _pool_body(table_hbm, idx_hbm, w_hbm, out_hbm,
               idx_v, w_v, rows0, rows1, acc, sem0, sem1):
    wid = lax.axis_index("s") * 2 + lax.axis_index("c")

    def roi_body(i, carry):
        n = wid * _RPW + i
        pltpu.sync_copy(idx_hbm.at[n], idx_v)
        pltpu.sync_copy(w_hbm.at[n], w_v)
        handles = [pltpu.async_copy(table_hbm.at[idx_v.at[0]], rows0, sem0),
                   None]
        for k in range(_NCHUNK):
            cur = k % 2
            handles[cur].wait()
            if k + 1 < _NCHUNK:
                nxt = (k + 1) % 2
                handles[nxt] = pltpu.async_copy(
                    table_hbm.at[idx_v.at[k + 1]],
                    rows1 if nxt else rows0,
                    sem1 if nxt else sem0)
            buf = rows1 if cur else rows0

            def bin_body(bb, c3_, _k=k, _buf=buf):
                base = bb * 16
                wv = w_v[pl.ds((_k * _CHUNK + base) * 16, 16)]
                accs = [wv * _buf[base, pl.ds(cc * 16, 16)]
                        for cc in range(16)]
                for rr in range(1, 16):
                    wv = w_v[pl.ds((_k * _CHUNK + base + rr) * 16, 16)]
                    for cc in range(16):
                        accs[cc] = accs[cc] + wv * _buf[base + rr,
                                                        pl.ds(cc * 16, 16)]
                b = _k * 7 + bb
                for cc in range(16):
                    acc[b, pl.ds(cc * 16, 16)] = accs[cc]
                return c3_
            lax.fori_loop(0, _CHUNK // 16, bin_body, 0)

        pltpu.sync_copy(acc, out_hbm.at[n])
        return carry
    lax.fori_loop(0, _RPW, roi_body, 0)


def _make_pool():
    return functools.partial(
        pl.kernel,
        mesh=plsc.VectorSubcoreMesh(core_axis_name="c", subcore_axis_name="s"),
        out_type=jax.ShapeDtypeStruct((_N, _OS * _OS, _C), jnp.float32),
        scratch_types=[
            pltpu.VMEM((_NCHUNK, _CHUNK), jnp.int32),
            pltpu.VMEM((_RPR * 16,), jnp.float32),
            pltpu.VMEM((_CHUNK, _C), jnp.float32),
            pltpu.VMEM((_CHUNK, _C), jnp.float32),
            pltpu.VMEM((_OS * _OS, _C), jnp.float32),
            pltpu.SemaphoreType.DMA,
            pltpu.SemaphoreType.DMA,
        ],
    )(_pool_body)


def _rows(x):
    b, c, h, w = x.shape
    return jnp.transpose(x, (0, 2, 3, 1)).reshape(b * h * w, c)


def kernel(x_p2, x_p3, x_p4, x_p5, boxes0, boxes1):
    table = jnp.concatenate(
        [_rows(x_p2), _rows(x_p3), _rows(x_p4), _rows(x_p5)], axis=0)
    i00, i01, i10, i11, w00, w01, w10, w11 = _make_grid()(boxes0, boxes1)
    idx = jnp.stack([i00, i01, i10, i11], axis=-1).reshape(_N, _NCHUNK, _CHUNK)
    w = jnp.stack([w00, w01, w10, w11], axis=-1).reshape(_N, _RPR)
    w = jnp.broadcast_to(w[:, :, None], (_N, _RPR, 16)).reshape(_N, _RPR * 16)
    out = _make_pool()(table, idx, w)
    return jnp.transpose(out.reshape(_N, _OS, _OS, _C), (0, 3, 1, 2))


# speedup vs baseline: 19.6960x; 1.2625x over previous
import functools

import jax
import jax.numpy as jnp
from jax import lax
from jax.experimental import pallas as pl
from jax.experimental.pallas import tpu as pltpu
from jax.experimental.pallas import tpu_sc as plsc

_OS = 7
_SR = 2
_N = 512
_C = 256
_NS = _OS * _OS * _SR * _SR
_RPR = _NS * 4
_NCHUNK = 7
_CHUNK = _RPR // _NCHUNK
_NW = 32
_RPW = _N // _NW
_EPS = 2.220446049250313e-16

_OFFS = (0, 131072, 163840, 172032)
_HS = (256, 128, 64, 32)


def _grid_body(b0_ref, b1_ref, i00, i01, i10, i11, w00, w01, w10, w11,
               lvl_ref):
    boxes = jnp.concatenate([b0_ref[...], b1_ref[...]], axis=0)
    c0 = boxes[:, 0:1]
    c1 = boxes[:, 1:2]
    c2 = boxes[:, 2:3]
    c3 = boxes[:, 3:4]
    t = jnp.sqrt((c2 - c0) * (c3 - c1)) / 224.0 + _EPS
    lvl = ((t >= 0.5).astype(jnp.int32) + (t >= 1.0).astype(jnp.int32)
           + (t >= 2.0).astype(jnp.int32))

    def bylvl(v0, v1, v2, v3, dtype):
        return jnp.where(
            lvl == 0, jnp.array(v0, dtype),
            jnp.where(lvl == 1, jnp.array(v1, dtype),
                      jnp.where(lvl == 2, jnp.array(v2, dtype),
                                jnp.array(v3, dtype))))

    scale = bylvl(0.25, 0.125, 0.0625, 0.03125, jnp.float32)
    hi = bylvl(*_HS, jnp.int32)
    hw = bylvl(256 * 256, 128 * 128, 64 * 64, 32 * 32, jnp.int32)
    ridx = lax.broadcasted_iota(jnp.int32, (_N, 1), 0)
    base = jnp.where(ridx >= 256, hw, 0)
    lvl_ref[...] = lvl

    x1 = c0 * scale
    y1 = c1 * scale
    x2 = c2 * scale
    y2 = c3 * scale
    bin_w = jnp.maximum(x2 - x1, 1.0) / _OS
    bin_h = jnp.maximum(y2 - y1, 1.0) / _OS

    s = lax.broadcasted_iota(jnp.int32, (_N, _NS), 1)
    phf = (s // 28).astype(jnp.float32)
    pwf = ((s // 4) % _OS).astype(jnp.float32)
    iyf = ((s % 4) // 2).astype(jnp.float32)
    ixf = (s % 2).astype(jnp.float32)

    y = y1 + phf * bin_h + (iyf + 0.5) * bin_h / _SR
    x = x1 + pwf * bin_w + (ixf + 0.5) * bin_w / _SR
    hf = hi.astype(jnp.float32)
    valid = (y > -1.0) & (y < hf) & (x > -1.0) & (x < hf)
    y = jnp.clip(y, 0.0, hf - 1.0)
    x = jnp.clip(x, 0.0, hf - 1.0)
    y0f = jnp.floor(y)
    x0f = jnp.floor(x)
    ly = y - y0f
    lx = x - x0f
    y0 = y0f.astype(jnp.int32)
    x0 = x0f.astype(jnp.int32)
    yb = jnp.minimum(y0 + 1, hi - 1)
    xb = jnp.minimum(x0 + 1, hi - 1)
    vw = jnp.where(valid, 0.25, 0.0)
    w00[...] = (1.0 - ly) * (1.0 - lx) * vw
    w01[...] = (1.0 - ly) * lx * vw
    w10[...] = ly * (1.0 - lx) * vw
    w11[...] = ly * lx * vw
    i00[...] = base + y0 * hi + x0
    i01[...] = base + y0 * hi + xb
    i10[...] = base + yb * hi + x0
    i11[...] = base + yb * hi + xb


def _make_grid():
    sd = jax.ShapeDtypeStruct
    return pl.pallas_call(
        _grid_body,
        out_shape=(
            sd((_N, _NS), jnp.int32), sd((_N, _NS), jnp.int32),
            sd((_N, _NS), jnp.int32), sd((_N, _NS), jnp.int32),
            sd((_N, _NS), jnp.float32), sd((_N, _NS), jnp.float32),
            sd((_N, _NS), jnp.float32), sd((_N, _NS), jnp.float32),
            sd((_N, 1), jnp.int32),
        ),
    )


def _pool_body(t2_hbm, t3_hbm, t4_hbm, t5_hbm, idx_hbm, w_hbm, lvl_hbm,
               out_hbm, idx_v, w_v, lvl_v, rows0, rows1, acc, sem0, sem1):
    wid = lax.axis_index("s") * 2 + lax.axis_index("c")
    tables = (t2_hbm, t3_hbm, t4_hbm, t5_hbm)
    pltpu.sync_copy(lvl_hbm.at[pl.ds(wid * _RPW, _RPW)], lvl_v)
    _lvlvec = lvl_v[...]
    _svals = [_lvlvec[j] for j in range(_RPW)]

    def issue_gather(lvl_s, kk, dst, sem):
        for lv in range(4):
            @pl.when(lvl_s == lv)
            def _(_t=tables[lv]):
                pltpu.async_copy(_t.at[idx_v.at[kk]], dst, sem)
        return pltpu.make_async_copy(t2_hbm.at[idx_v.at[kk]], dst, sem)

    def roi_body(i, carry):
        n = wid * _RPW + i
        lvl_s = _svals[0]
        for j in range(1, _RPW):
            lvl_s = jnp.where(i == j, _svals[j], lvl_s)
        pltpu.sync_copy(idx_hbm.at[n], idx_v)
        pltpu.sync_copy(w_hbm.at[n], w_v)
        handles = [issue_gather(lvl_s, 0, rows0, sem0), None]
        for k in range(_NCHUNK):
            cur = k % 2
            handles[cur].wait()
            if k + 1 < _NCHUNK:
                nxt = (k + 1) % 2
                handles[nxt] = issue_gather(
                    lvl_s, k + 1,
                    rows1 if nxt else rows0,
                    sem1 if nxt else sem0)
            buf = rows1 if cur else rows0

            def bin_body(bb, c3_, _k=k, _buf=buf):
                base = bb * 16
                wv = w_v[pl.ds((_k * _CHUNK + base) * 16, 16)]
                accs = [wv * _buf[base, pl.ds(cc * 16, 16)]
                        for cc in range(16)]
                for rr in range(1, 16):
                    wv = w_v[pl.ds((_k * _CHUNK + base + rr) * 16, 16)]
                    for cc in range(16):
                        accs[cc] = accs[cc] + wv * _buf[base + rr,
                                                        pl.ds(cc * 16, 16)]
                b = _k * 7 + bb
                for cc in range(16):
                    acc[b, pl.ds(cc * 16, 16)] = accs[cc]
                return c3_
            lax.fori_loop(0, _CHUNK // 16, bin_body, 0)

        pltpu.sync_copy(acc, out_hbm.at[n])
        return carry
    lax.fori_loop(0, _RPW, roi_body, 0)


def _make_pool():
    return functools.partial(
        pl.kernel,
        mesh=plsc.VectorSubcoreMesh(core_axis_name="c", subcore_axis_name="s"),
        out_type=jax.ShapeDtypeStruct((_N, _OS * _OS, _C), jnp.float32),
        scratch_types=[
            pltpu.VMEM((_NCHUNK, _CHUNK), jnp.int32),
            pltpu.VMEM((_RPR * 16,), jnp.float32),
            pltpu.VMEM((_RPW,), jnp.int32),
            pltpu.VMEM((_CHUNK, _C), jnp.float32),
            pltpu.VMEM((_CHUNK, _C), jnp.float32),
            pltpu.VMEM((_OS * _OS, _C), jnp.float32),
            pltpu.SemaphoreType.DMA,
            pltpu.SemaphoreType.DMA,
        ],
    )(_pool_body)


def _rows(x):
    b, c, h, w = x.shape
    return jnp.transpose(x, (0, 2, 3, 1)).reshape(b * h * w, c)


def kernel(x_p2, x_p3, x_p4, x_p5, boxes0, boxes1):
    i00, i01, i10, i11, w00, w01, w10, w11, lvl = _make_grid()(boxes0, boxes1)
    idx = jnp.stack([i00, i01, i10, i11], axis=-1).reshape(_N, _NCHUNK, _CHUNK)
    w = jnp.stack([w00, w01, w10, w11], axis=-1).reshape(_N, _RPR)
    w = jnp.broadcast_to(w[:, :, None], (_N, _RPR, 16)).reshape(_N, _RPR * 16)
    out = _make_pool()(_rows(x_p2), _rows(x_p3), _rows(x_p4), _rows(x_p5),
                       idx, w, lvl.reshape(_N))
    return jnp.transpose(out.reshape(_N, _OS, _OS, _C), (0, 3, 1, 2))


# speedup vs baseline: 21.0479x; 1.0686x over previous
import functools

import jax
import jax.numpy as jnp
from jax import lax
from jax.experimental import pallas as pl
from jax.experimental.pallas import tpu as pltpu
from jax.experimental.pallas import tpu_sc as plsc

_OS = 7
_SR = 2
_N = 512
_C = 256
_NS = _OS * _OS * _SR * _SR
_RPR = _NS * 4
_NCHUNK = 7
_CHUNK = _RPR // _NCHUNK
_NW = 32
_RPW = _N // _NW
_EPS = 2.220446049250313e-16

_OFFS = (0, 131072, 163840, 172032)
_HS = (256, 128, 64, 32)


def _grid_body(b0_ref, b1_ref, i00, i01, i10, i11, w00, w01, w10, w11,
               lvl_ref):
    boxes = jnp.concatenate([b0_ref[...], b1_ref[...]], axis=0)
    c0 = boxes[:, 0:1]
    c1 = boxes[:, 1:2]
    c2 = boxes[:, 2:3]
    c3 = boxes[:, 3:4]
    t = jnp.sqrt((c2 - c0) * (c3 - c1)) / 224.0 + _EPS
    lvl = ((t >= 0.5).astype(jnp.int32) + (t >= 1.0).astype(jnp.int32)
           + (t >= 2.0).astype(jnp.int32))

    def bylvl(v0, v1, v2, v3, dtype):
        return jnp.where(
            lvl == 0, jnp.array(v0, dtype),
            jnp.where(lvl == 1, jnp.array(v1, dtype),
                      jnp.where(lvl == 2, jnp.array(v2, dtype),
                                jnp.array(v3, dtype))))

    scale = bylvl(0.25, 0.125, 0.0625, 0.03125, jnp.float32)
    hi = bylvl(*_HS, jnp.int32)
    hw = bylvl(256 * 256, 128 * 128, 64 * 64, 32 * 32, jnp.int32)
    ridx = lax.broadcasted_iota(jnp.int32, (_N, 1), 0)
    base = jnp.where(ridx >= 256, hw, 0)
    lvl_ref[...] = lvl

    x1 = c0 * scale
    y1 = c1 * scale
    x2 = c2 * scale
    y2 = c3 * scale
    bin_w = jnp.maximum(x2 - x1, 1.0) / _OS
    bin_h = jnp.maximum(y2 - y1, 1.0) / _OS

    s = lax.broadcasted_iota(jnp.int32, (_N, _NS), 1)
    phf = (s // 28).astype(jnp.float32)
    pwf = ((s // 4) % _OS).astype(jnp.float32)
    iyf = ((s % 4) // 2).astype(jnp.float32)
    ixf = (s % 2).astype(jnp.float32)

    y = y1 + phf * bin_h + (iyf + 0.5) * bin_h / _SR
    x = x1 + pwf * bin_w + (ixf + 0.5) * bin_w / _SR
    hf = hi.astype(jnp.float32)
    valid = (y > -1.0) & (y < hf) & (x > -1.0) & (x < hf)
    y = jnp.clip(y, 0.0, hf - 1.0)
    x = jnp.clip(x, 0.0, hf - 1.0)
    y0f = jnp.floor(y)
    x0f = jnp.floor(x)
    ly = y - y0f
    lx = x - x0f
    y0 = y0f.astype(jnp.int32)
    x0 = x0f.astype(jnp.int32)
    yb = jnp.minimum(y0 + 1, hi - 1)
    xb = jnp.minimum(x0 + 1, hi - 1)
    vw = jnp.where(valid, 0.25, 0.0)
    w00[...] = (1.0 - ly) * (1.0 - lx) * vw
    w01[...] = (1.0 - ly) * lx * vw
    w10[...] = ly * (1.0 - lx) * vw
    w11[...] = ly * lx * vw
    i00[...] = base + y0 * hi + x0
    i01[...] = base + y0 * hi + xb
    i10[...] = base + yb * hi + x0
    i11[...] = base + yb * hi + xb


def _make_grid():
    sd = jax.ShapeDtypeStruct
    return pl.pallas_call(
        _grid_body,
        out_shape=(
            sd((_N, _NS), jnp.int32), sd((_N, _NS), jnp.int32),
            sd((_N, _NS), jnp.int32), sd((_N, _NS), jnp.int32),
            sd((_N, _NS), jnp.float32), sd((_N, _NS), jnp.float32),
            sd((_N, _NS), jnp.float32), sd((_N, _NS), jnp.float32),
            sd((_N, 1), jnp.int32),
        ),
    )


def _pool_body(t2_hbm, t3_hbm, t4_hbm, t5_hbm, idx_hbm, w_hbm, lvl_hbm,
               out_hbm, idx_v, w_v, lvl_v, rows0, rows1, acc, sem0, sem1):
    wid = lax.axis_index("s") * 2 + lax.axis_index("c")
    tables = (t2_hbm, t3_hbm, t4_hbm, t5_hbm)
    pltpu.sync_copy(lvl_hbm.at[pl.ds(wid * _RPW, _RPW)], lvl_v)
    _lvlvec = lvl_v[...]
    _svals = [_lvlvec[j] for j in range(_RPW)]

    def issue_gather(lvl_s, kk, dst, sem):
        for lv in range(4):
            @pl.when(lvl_s == lv)
            def _(_t=tables[lv]):
                pltpu.async_copy(_t.at[idx_v.at[kk]], dst, sem)
        return pltpu.make_async_copy(t2_hbm.at[idx_v.at[kk]], dst, sem)

    def roi_body(i, carry):
        n = wid * _RPW + i
        lvl_s = _svals[0]
        for j in range(1, _RPW):
            lvl_s = jnp.where(i == j, _svals[j], lvl_s)
        pltpu.sync_copy(idx_hbm.at[n], idx_v)
        pltpu.sync_copy(w_hbm.at[n], w_v)
        handles = [issue_gather(lvl_s, 0, rows0, sem0), None]
        for k in range(_NCHUNK):
            cur = k % 2
            handles[cur].wait()
            if k + 1 < _NCHUNK:
                nxt = (k + 1) % 2
                handles[nxt] = issue_gather(
                    lvl_s, k + 1,
                    rows1 if nxt else rows0,
                    sem1 if nxt else sem0)
            buf = rows1 if cur else rows0

            def bin_body(bb, c3_, _k=k, _buf=buf):
                base = bb * 16
                w16 = w_v[pl.ds(_k * _CHUNK + base, 16)]
                accs = [w16[0] * _buf[base, pl.ds(cc * 16, 16)]
                        for cc in range(16)]
                for rr in range(1, 16):
                    for cc in range(16):
                        accs[cc] = accs[cc] + w16[rr] * _buf[base + rr,
                                                             pl.ds(cc * 16, 16)]
                b = _k * 7 + bb
                for cc in range(16):
                    acc[b, pl.ds(cc * 16, 16)] = accs[cc]
                return c3_
            lax.fori_loop(0, _CHUNK // 16, bin_body, 0)

        pltpu.sync_copy(acc, out_hbm.at[n])
        return carry
    lax.fori_loop(0, _RPW, roi_body, 0)


def _make_pool():
    return functools.partial(
        pl.kernel,
        mesh=plsc.VectorSubcoreMesh(core_axis_name="c", subcore_axis_name="s"),
        out_type=jax.ShapeDtypeStruct((_N, _OS * _OS, _C), jnp.float32),
        scratch_types=[
            pltpu.VMEM((_NCHUNK, _CHUNK), jnp.int32),
            pltpu.VMEM((_RPR,), jnp.float32),
            pltpu.VMEM((_RPW,), jnp.int32),
            pltpu.VMEM((_CHUNK, _C), jnp.float32),
            pltpu.VMEM((_CHUNK, _C), jnp.float32),
            pltpu.VMEM((_OS * _OS, _C), jnp.float32),
            pltpu.SemaphoreType.DMA,
            pltpu.SemaphoreType.DMA,
        ],
    )(_pool_body)


def _rows(x):
    b, c, h, w = x.shape
    return jnp.transpose(x, (0, 2, 3, 1)).reshape(b * h * w, c)


def kernel(x_p2, x_p3, x_p4, x_p5, boxes0, boxes1):
    i00, i01, i10, i11, w00, w01, w10, w11, lvl = _make_grid()(boxes0, boxes1)
    idx = jnp.stack([i00, i01, i10, i11], axis=-1).reshape(_N, _NCHUNK, _CHUNK)
    w = jnp.stack([w00, w01, w10, w11], axis=-1).reshape(_N, _RPR)
    out = _make_pool()(_rows(x_p2), _rows(x_p3), _rows(x_p4), _rows(x_p5),
                       idx, w, lvl.reshape(_N))
    return jnp.transpose(out.reshape(_N, _OS, _OS, _C), (0, 3, 1, 2))
